# baseline (speedup 1.0000x reference)
import jax
import jax.numpy as jnp
from jax.experimental import pallas as pl

ROWS = 128
D = 32768
N_BISECT = 32


def _tc_body(x_ref, y_ref):
    x = x_ref[...]
    m = jnp.max(x, axis=1, keepdims=True)
    xs = (x - m) * 0.5
    lo = jnp.full((x.shape[0], 1), -1.0, jnp.float32)
    hi = jnp.zeros((x.shape[0], 1), jnp.float32)
    for _ in range(N_BISECT):
        mid = 0.5 * (lo + hi)
        f = jnp.sum(jnp.maximum(xs - mid, 0.0) ** 2, axis=1, keepdims=True)
        ge = f >= 1.0
        lo = jnp.where(ge, mid, lo)
        hi = jnp.where(ge, hi, mid)
    tau = 0.5 * (lo + hi)
    r = jnp.maximum(xs - tau, 0.0)
    y_ref[...] = r * r


@jax.jit
def kernel(X):
    return pl.pallas_call(
        _tc_body,
        out_shape=jax.ShapeDtypeStruct((ROWS, D), jnp.float32),
        grid=(ROWS // 8,),
        in_specs=[pl.BlockSpec((8, D), lambda i: (i, 0))],
        out_specs=pl.BlockSpec((8, D), lambda i: (i, 0)),
    )(X)


# speedup vs baseline: 2.7353x; 2.7353x over previous
import jax
import jax.numpy as jnp
from jax import lax
from jax.experimental import pallas as pl
from jax.experimental.pallas import tpu as pltpu
from jax.experimental.pallas import tpu_sc as plsc

L = 16
ROWS = 128
D = 32768
CHUNKS = D // L
NC, NS = 2, 16
NW = NC * NS
K_TC = 32
RPW = (ROWS - K_TC) // NW
N_BISECT = 32


def _take(v, idx):
    return v.at[idx].get(mode="promise_in_bounds")


def _all_max(v, iota):
    for s in (8, 4, 2, 1):
        v = jnp.maximum(v, _take(v, iota ^ s))
    return v


def _all_sum(v, iota):
    for s in (8, 4, 2, 1):
        v = v + _take(v, iota ^ s)
    return v


def _sc_body(x_hbm, y_hbm, row0_v, row1_v, surv_v, sin0, sin1, sout0, sout1):
    cid = lax.axis_index("c")
    sid = lax.axis_index("s")
    wid = sid * NC + cid
    iota = lax.iota(jnp.int32, L)
    sin = (sin0, sin1)
    sout = (sout0, sout1)
    bufs = (row0_v, row1_v)

    def compute(row_v):
        @plsc.parallel_loop(0, CHUNKS, unroll=8,
                            carry=jnp.full((L,), -jnp.inf, jnp.float32))
        def macc(i, acc):
            return jnp.maximum(acc, row_v[pl.ds(i * L, L)])

        m = _all_max(macc, iota)
        thresh = m - 2.0

        @plsc.parallel_loop(0, CHUNKS, unroll=8,
                            carry=jnp.zeros((L,), jnp.int32))
        def off(i, off_c):
            x = row_v[pl.ds(i * L, L)]
            mask = x > thresh
            xs = (x - m) * 0.5
            sk, _sv, _om = plsc.sort_key_val(xs, xs, mask=mask,
                                             descending=True)
            cnt = plsc.all_reduce_population_count(mask)
            idx = jnp.where(iota < cnt, off_c + iota,
                            jnp.full((L,), D + L - 1, jnp.int32))
            plsc.store_scatter(surv_v, [idx], sk)
            return off_c + cnt

        nsurv = off[0]
        surv_v[pl.ds(nsurv, L)] = jnp.full((L,), -4.0, jnp.float32)
        nchunks = (nsurv + L - 1) // L

        def bis_body(it, lohi):
            lo, hi = lohi
            mid = 0.5 * (lo + hi)

            @plsc.parallel_loop(0, nchunks, unroll=8,
                                carry=jnp.zeros((L,), jnp.float32))
            def fv(i, acc):
                r = jnp.maximum(surv_v[pl.ds(i * L, L)] - mid, 0.0)
                return acc + r * r

            ge = _all_sum(fv, iota) >= 1.0
            return (jnp.where(ge, mid, lo), jnp.where(ge, hi, mid))

        lo, hi = lax.fori_loop(
            0, N_BISECT, bis_body,
            (jnp.full((L,), -1.0, jnp.float32), jnp.zeros((L,), jnp.float32)))
        tau = 0.5 * (lo + hi)
        c = m + 2.0 * tau

        @plsc.parallel_loop(0, CHUNKS, unroll=8)
        def _(i):
            r = jnp.maximum((row_v[pl.ds(i * L, L)] - c) * 0.5, 0.0)
            row_v[pl.ds(i * L, L)] = r * r

    base = K_TC + wid * RPW
    copies_out = [None] * RPW
    copy_in = [None] * RPW
    copy_in[0] = pltpu.async_copy(x_hbm.at[base], bufs[0], sin[0])
    for j in range(RPW):
        b = j % 2
        if j + 1 < RPW:
            if j - 1 >= 0:
                copies_out[j - 1].wait()
            copy_in[j + 1] = pltpu.async_copy(
                x_hbm.at[base + j + 1], bufs[(j + 1) % 2], sin[(j + 1) % 2])
        copy_in[j].wait()
        compute(bufs[b])
        copies_out[j] = pltpu.async_copy(bufs[b], y_hbm.at[base + j], sout[b])
    copies_out[RPW - 2].wait()
    copies_out[RPW - 1].wait()


def _tc_body(x_ref, y_ref):
    x = x_ref[...]
    m = jnp.max(x, axis=1, keepdims=True)
    xs = (x - m) * 0.5
    lo = jnp.full((x.shape[0], 1), -1.0, jnp.float32)
    hi = jnp.zeros((x.shape[0], 1), jnp.float32)
    for _ in range(N_BISECT):
        mid = 0.5 * (lo + hi)
        f = jnp.sum(jnp.maximum(xs - mid, 0.0) ** 2, axis=1, keepdims=True)
        ge = f >= 1.0
        lo = jnp.where(ge, mid, lo)
        hi = jnp.where(ge, hi, mid)
    tau = 0.5 * (lo + hi)
    r = jnp.maximum(xs - tau, 0.0)
    y_ref[...] = r * r


@jax.jit
def kernel(X):
    sc_k = pl.kernel(
        _sc_body,
        out_type=jax.ShapeDtypeStruct((ROWS, D), jnp.float32),
        mesh=plsc.VectorSubcoreMesh(core_axis_name="c", subcore_axis_name="s"),
        compiler_params=pltpu.CompilerParams(needs_layout_passes=False),
        scratch_types=[
            pltpu.VMEM((D,), jnp.float32),
            pltpu.VMEM((D,), jnp.float32),
            pltpu.VMEM((D + L,), jnp.float32),
            pltpu.SemaphoreType.DMA,
            pltpu.SemaphoreType.DMA,
            pltpu.SemaphoreType.DMA,
            pltpu.SemaphoreType.DMA,
        ],
    )
    y_sc = sc_k(X)
    y_tc = pl.pallas_call(
        _tc_body,
        out_shape=jax.ShapeDtypeStruct((K_TC, D), jnp.float32),
        grid=(K_TC // 8,),
        in_specs=[pl.BlockSpec((8, D), lambda i: (i, 0))],
        out_specs=pl.BlockSpec((8, D), lambda i: (i, 0)),
    )(X)
    return jax.lax.dynamic_update_slice(y_sc, y_tc, (0, 0))


# speedup vs baseline: 2.9021x; 1.0610x over previous
import jax
import jax.numpy as jnp
from jax import lax
from jax.experimental import pallas as pl
from jax.experimental.pallas import tpu as pltpu
from jax.experimental.pallas import tpu_sc as plsc

L = 16
ROWS = 128
D = 32768
CHUNKS = D // L
NC, NS = 2, 16
NW = NC * NS
RPW = ROWS // NW
N_BISECT = 28


def _take(v, idx):
    return v.at[idx].get(mode="promise_in_bounds")


def _all_max(v, iota):
    for s in (8, 4, 2, 1):
        v = jnp.maximum(v, _take(v, iota ^ s))
    return v


def _all_sum(v, iota):
    for s in (8, 4, 2, 1):
        v = v + _take(v, iota ^ s)
    return v


def _prefix_sum(v, iota):
    for s in (1, 2, 4, 8):
        shifted = _take(v, jnp.maximum(iota - s, 0))
        v = v + jnp.where(iota >= s, shifted, 0)
    return v


def _tsallis_body(x_hbm, y_hbm, row0_v, row1_v, surv_v, sin0, sin1, sout0,
                  sout1):
    cid = lax.axis_index("c")
    sid = lax.axis_index("s")
    wid = sid * NC + cid
    iota = lax.iota(jnp.int32, L)
    fifteen = jnp.full((L,), 15, jnp.int32)
    sin = (sin0, sin1)
    sout = (sout0, sout1)
    bufs = (row0_v, row1_v)

    def compute(row_v):
        @plsc.parallel_loop(0, CHUNKS, unroll=16,
                            carry=jnp.full((L,), -jnp.inf, jnp.float32))
        def macc(i, acc):
            return jnp.maximum(acc, row_v[pl.ds(i * L, L)])

        m = _all_max(macc, iota)
        thresh = m - 2.0

        @plsc.parallel_loop(0, CHUNKS, unroll=8,
                            carry=jnp.zeros((L,), jnp.int32))
        def off(i, off_c):
            x = row_v[pl.ds(i * L, L)]
            mask = x > thresh
            xs = (x - m) * 0.5
            sk, _sv, _om = plsc.sort_key_val(xs, xs, mask=mask,
                                             descending=True)
            cnt = plsc.all_reduce_population_count(mask)
            idx = jnp.where(iota < cnt, off_c + iota,
                            jnp.full((L,), D + L - 1, jnp.int32))
            plsc.store_scatter(surv_v, [idx], sk)
            return off_c + cnt

        nsurv = off[0]
        surv_v[pl.ds(nsurv, L)] = jnp.full((L,), -4.0, jnp.float32)
        nchunks = (nsurv + L - 1) // L

        def bis_body(it, lohi):
            lo, hi = lohi
            mid = 0.5 * (lo + hi)

            @plsc.parallel_loop(0, nchunks, unroll=8,
                                carry=jnp.zeros((L,), jnp.float32))
            def fv(i, acc):
                r = jnp.maximum(surv_v[pl.ds(i * L, L)] - mid, 0.0)
                return acc + r * r

            ge = _all_sum(fv, iota) >= 1.0
            return (jnp.where(ge, mid, lo), jnp.where(ge, hi, mid))

        lo, hi = lax.fori_loop(
            0, N_BISECT, bis_body,
            (jnp.full((L,), -1.0, jnp.float32), jnp.zeros((L,), jnp.float32)))
        tau = 0.5 * (lo + hi)
        c = m + 2.0 * tau

        @plsc.parallel_loop(0, CHUNKS, unroll=16)
        def _(i):
            r = jnp.maximum((row_v[pl.ds(i * L, L)] - c) * 0.5, 0.0)
            row_v[pl.ds(i * L, L)] = r * r

    base = wid * RPW
    copies_out = [None] * RPW
    copy_in = [None] * RPW
    copy_in[0] = pltpu.async_copy(x_hbm.at[base], bufs[0], sin[0])
    for j in range(RPW):
        b = j % 2
        if j + 1 < RPW:
            if j - 1 >= 0:
                copies_out[j - 1].wait()
            copy_in[j + 1] = pltpu.async_copy(
                x_hbm.at[base + j + 1], bufs[(j + 1) % 2], sin[(j + 1) % 2])
        copy_in[j].wait()
        compute(bufs[b])
        copies_out[j] = pltpu.async_copy(bufs[b], y_hbm.at[base + j], sout[b])
    copies_out[RPW - 2].wait()
    copies_out[RPW - 1].wait()


@jax.jit
def kernel(X):
    k = pl.kernel(
        _tsallis_body,
        out_type=jax.ShapeDtypeStruct((ROWS, D), jnp.float32),
        mesh=plsc.VectorSubcoreMesh(core_axis_name="c", subcore_axis_name="s"),
        compiler_params=pltpu.CompilerParams(needs_layout_passes=False),
        scratch_types=[
            pltpu.VMEM((D,), jnp.float32),
            pltpu.VMEM((D,), jnp.float32),
            pltpu.VMEM((D + L,), jnp.float32),
            pltpu.SemaphoreType.DMA,
            pltpu.SemaphoreType.DMA,
            pltpu.SemaphoreType.DMA,
            pltpu.SemaphoreType.DMA,
        ],
    )
    return k(X)


# speedup vs baseline: 3.0162x; 1.0393x over previous
import jax
import jax.numpy as jnp
from jax import lax
from jax.experimental import pallas as pl
from jax.experimental.pallas import tpu as pltpu
from jax.experimental.pallas import tpu_sc as plsc

L = 16
ROWS = 128
D = 32768
CHUNKS = D // L
NC, NS = 2, 16
NW = NC * NS
RPW = ROWS // NW
N_BISECT = 18


def _take(v, idx):
    return v.at[idx].get(mode="promise_in_bounds")


def _all_max(v, iota):
    for s in (8, 4, 2, 1):
        v = jnp.maximum(v, _take(v, iota ^ s))
    return v


def _all_sum(v, iota):
    for s in (8, 4, 2, 1):
        v = v + _take(v, iota ^ s)
    return v


def _prefix_sum(v, iota):
    for s in (1, 2, 4, 8):
        shifted = _take(v, jnp.maximum(iota - s, 0))
        v = v + jnp.where(iota >= s, shifted, 0)
    return v


def _tsallis_body(x_hbm, y_hbm, row0_v, row1_v, surv_v, sin0, sin1, sout0,
                  sout1):
    cid = lax.axis_index("c")
    sid = lax.axis_index("s")
    wid = sid * NC + cid
    iota = lax.iota(jnp.int32, L)
    fifteen = jnp.full((L,), 15, jnp.int32)
    sin = (sin0, sin1)
    sout = (sout0, sout1)
    bufs = (row0_v, row1_v)

    def compute(row_v):
        @plsc.parallel_loop(0, CHUNKS, unroll=16,
                            carry=jnp.full((L,), -jnp.inf, jnp.float32))
        def macc(i, acc):
            return jnp.maximum(acc, row_v[pl.ds(i * L, L)])

        m = _all_max(macc, iota)
        thresh = m - 2.0

        @plsc.parallel_loop(0, CHUNKS, unroll=8,
                            carry=jnp.zeros((L,), jnp.int32))
        def off(i, off_c):
            x = row_v[pl.ds(i * L, L)]
            mask = x > thresh
            xs = (x - m) * 0.5
            sk, _sv, _om = plsc.sort_key_val(xs, xs, mask=mask,
                                             descending=True)
            cnt = plsc.all_reduce_population_count(mask)
            idx = jnp.where(iota < cnt, off_c + iota,
                            jnp.full((L,), D + L - 1, jnp.int32))
            plsc.store_scatter(surv_v, [idx], sk)
            return off_c + cnt

        nsurv = off[0]
        surv_v[pl.ds(nsurv, L)] = jnp.full((L,), -4.0, jnp.float32)
        nchunks = (nsurv + L - 1) // L

        third = jnp.float32(1.0 / 3.0)

        def bis_body(it, lohi):
            lo, hi = lohi
            dt = (hi - lo) * third
            m1 = lo + dt
            m2 = hi - dt

            @plsc.parallel_loop(0, nchunks, unroll=8,
                                carry=(jnp.zeros((L,), jnp.float32),
                                       jnp.zeros((L,), jnp.float32)))
            def fv(i, acc):
                a1, a2 = acc
                v = surv_v[pl.ds(i * L, L)]
                r1 = jnp.maximum(v - m1, 0.0)
                r2 = jnp.maximum(v - m2, 0.0)
                return (a1 + r1 * r1, a2 + r2 * r2)

            ge1 = _all_sum(fv[0], iota) >= 1.0
            ge2 = _all_sum(fv[1], iota) >= 1.0
            lo = jnp.where(ge2, m2, jnp.where(ge1, m1, lo))
            hi = jnp.where(ge1, jnp.where(ge2, hi, m2), m1)
            return (lo, hi)

        lo, hi = lax.fori_loop(
            0, N_BISECT, bis_body,
            (jnp.full((L,), -1.0, jnp.float32), jnp.zeros((L,), jnp.float32)))
        tau = 0.5 * (lo + hi)
        c = m + 2.0 * tau

        @plsc.parallel_loop(0, CHUNKS, unroll=16)
        def _(i):
            r = jnp.maximum((row_v[pl.ds(i * L, L)] - c) * 0.5, 0.0)
            row_v[pl.ds(i * L, L)] = r * r

    base = wid * RPW
    copies_out = [None] * RPW
    copy_in = [None] * RPW
    copy_in[0] = pltpu.async_copy(x_hbm.at[base], bufs[0], sin[0])
    for j in range(RPW):
        b = j % 2
        if j + 1 < RPW:
            if j - 1 >= 0:
                copies_out[j - 1].wait()
            copy_in[j + 1] = pltpu.async_copy(
                x_hbm.at[base + j + 1], bufs[(j + 1) % 2], sin[(j + 1) % 2])
        copy_in[j].wait()
        compute(bufs[b])
        copies_out[j] = pltpu.async_copy(bufs[b], y_hbm.at[base + j], sout[b])
    copies_out[RPW - 2].wait()
    copies_out[RPW - 1].wait()


@jax.jit
def kernel(X):
    k = pl.kernel(
        _tsallis_body,
        out_type=jax.ShapeDtypeStruct((ROWS, D), jnp.float32),
        mesh=plsc.VectorSubcoreMesh(core_axis_name="c", subcore_axis_name="s"),
        compiler_params=pltpu.CompilerParams(needs_layout_passes=False),
        scratch_types=[
            pltpu.VMEM((D,), jnp.float32),
            pltpu.VMEM((D,), jnp.float32),
            pltpu.VMEM((D + L,), jnp.float32),
            pltpu.SemaphoreType.DMA,
            pltpu.SemaphoreType.DMA,
            pltpu.SemaphoreType.DMA,
            pltpu.SemaphoreType.DMA,
        ],
    )
    return k(X)


# speedup vs baseline: 3.0189x; 1.0009x over previous
import jax
import jax.numpy as jnp
from jax import lax
from jax.experimental import pallas as pl
from jax.experimental.pallas import tpu as pltpu
from jax.experimental.pallas import tpu_sc as plsc

L = 16
ROWS = 128
D = 32768
CHUNKS = D // L
NC, NS = 2, 16
NW = NC * NS
RPW = ROWS // NW
N_BISECT = 18


def _take(v, idx):
    return v.at[idx].get(mode="promise_in_bounds")


def _all_max(v, iota):
    for s in (8, 4, 2, 1):
        v = jnp.maximum(v, _take(v, iota ^ s))
    return v


def _all_sum(v, iota):
    for s in (8, 4, 2, 1):
        v = v + _take(v, iota ^ s)
    return v


def _tsallis_body(x_hbm, y_hbm, row0_v, row1_v, surv_v, sin0, sin1, sout0,
                  sout1):
    cid = lax.axis_index("c")
    sid = lax.axis_index("s")
    wid = sid * NC + cid
    iota = lax.iota(jnp.int32, L)
    sin = (sin0, sin1)
    sout = (sout0, sout1)
    bufs = (row0_v, row1_v)

    def compute(row_v):
        @plsc.parallel_loop(0, CHUNKS, unroll=16,
                            carry=jnp.full((L,), -jnp.inf, jnp.float32))
        def macc(i, acc):
            return jnp.maximum(acc, row_v[pl.ds(i * L, L)])

        m = _all_max(macc, iota)
        thresh = m - 2.0

        @plsc.parallel_loop(0, CHUNKS, unroll=8,
                            carry=jnp.zeros((L,), jnp.int32))
        def off(i, off_c):
            x = row_v[pl.ds(i * L, L)]
            mask = x > thresh
            xs = (x - m) * 0.5
            sk, _sv, _om = plsc.sort_key_val(xs, xs, mask=mask,
                                             descending=True)
            cnt = plsc.all_reduce_population_count(mask)
            idx = jnp.where(iota < cnt, off_c + iota,
                            jnp.full((L,), D + L - 1, jnp.int32))
            plsc.store_scatter(surv_v, [idx], sk)
            return off_c + cnt

        nsurv = off[0]
        surv_v[pl.ds(nsurv, L)] = jnp.full((L,), -4.0, jnp.float32)
        nchunks = (nsurv + L - 1) // L

        third = jnp.float32(1.0 / 3.0)

        def bis_body(it, lohi):
            lo, hi = lohi
            dt = (hi - lo) * third
            m1 = lo + dt
            m2 = hi - dt

            @plsc.parallel_loop(0, nchunks, unroll=8,
                                carry=(jnp.zeros((L,), jnp.float32),
                                       jnp.zeros((L,), jnp.float32)))
            def fv(i, acc):
                a1, a2 = acc
                v = surv_v[pl.ds(i * L, L)]
                r1 = jnp.maximum(v - m1, 0.0)
                r2 = jnp.maximum(v - m2, 0.0)
                return (a1 + r1 * r1, a2 + r2 * r2)

            ge1 = _all_sum(fv[0], iota) >= 1.0
            ge2 = _all_sum(fv[1], iota) >= 1.0
            lo = jnp.where(ge2, m2, jnp.where(ge1, m1, lo))
            hi = jnp.where(ge1, jnp.where(ge2, hi, m2), m1)
            return (lo, hi)

        lo, hi = lax.fori_loop(
            0, N_BISECT, bis_body,
            (jnp.full((L,), -1.0, jnp.float32), jnp.zeros((L,), jnp.float32)))
        tau = 0.5 * (lo + hi)
        c = m + 2.0 * tau

        @plsc.parallel_loop(0, CHUNKS, unroll=16)
        def _(i):
            r = jnp.maximum((row_v[pl.ds(i * L, L)] - c) * 0.5, 0.0)
            row_v[pl.ds(i * L, L)] = r * r

    base = wid * RPW
    copies_out = [None] * RPW
    copy_in = [None] * RPW
    copy_in[0] = pltpu.async_copy(x_hbm.at[base], bufs[0], sin[0])
    for j in range(RPW):
        b = j % 2
        if j + 1 < RPW:
            if j - 1 >= 0:
                copies_out[j - 1].wait()
            copy_in[j + 1] = pltpu.async_copy(
                x_hbm.at[base + j + 1], bufs[(j + 1) % 2], sin[(j + 1) % 2])
        copy_in[j].wait()
        compute(bufs[b])
        copies_out[j] = pltpu.async_copy(bufs[b], y_hbm.at[base + j], sout[b])
    copies_out[RPW - 2].wait()
    copies_out[RPW - 1].wait()


@jax.jit
def kernel(X):
    k = pl.kernel(
        _tsallis_body,
        out_type=jax.ShapeDtypeStruct((ROWS, D), jnp.float32),
        mesh=plsc.VectorSubcoreMesh(core_axis_name="c", subcore_axis_name="s"),
        compiler_params=pltpu.CompilerParams(needs_layout_passes=False),
        scratch_types=[
            pltpu.VMEM((D,), jnp.float32),
            pltpu.VMEM((D,), jnp.float32),
            pltpu.VMEM((D + L,), jnp.float32),
            pltpu.SemaphoreType.DMA,
            pltpu.SemaphoreType.DMA,
            pltpu.SemaphoreType.DMA,
            pltpu.SemaphoreType.DMA,
        ],
    )
    return k(X)
